# Initial kernel scaffold; baseline (speedup 1.0000x reference)
import jax
import jax.numpy as jnp
from jax.experimental import pallas as pl


def kernel(x, Wr, Wu, Wg, Wd, Wu_s, Wg_s, Wd_s):
    raise NotImplementedError("write your pallas kernel here")


# speedup vs baseline: 1.3373x; 1.3373x over previous
import functools

import jax
import jax.numpy as jnp
from jax.experimental import pallas as pl
from jax.experimental.pallas import tpu as pltpu

DIM = 1024
NUM_EXPERTS = 8
N_ROUTED = 7
HIDDEN = 4096
ALPHA = 2.0
TOKENS = 64
HB = 2048
NH = HIDDEN // HB


def _dotT(a, b):
    return jax.lax.dot_general(a, b, (((1,), (1,)), ((), ())),
                               preferred_element_type=jnp.float32)


def _routed_body(x_ref, wr_ref, wu_ref, wg_ref, wd_ref, out_ref, scale_ref):
    j = pl.program_id(0)
    h = pl.program_id(1)

    @pl.when((j == 0) & (h == 0))
    def _init():
        logits = _dotT(x_ref[...], wr_ref[...])
        m = jnp.max(logits, axis=-1, keepdims=True)
        e = jnp.exp(logits - m)
        p = e / jnp.sum(e, axis=-1, keepdims=True)
        amax = jnp.argmax(logits, axis=-1)
        pmax = jnp.max(p, axis=-1)
        cols = jax.lax.broadcasted_iota(jnp.int32, (TOKENS, NUM_EXPERTS), 1)
        scale_ref[...] = jnp.where(cols == amax[:, None],
                                   ALPHA * pmax[:, None], 0.0)
        out_ref[...] = jnp.zeros_like(out_ref)

    x = x_ref[...]
    up = _dotT(x, wu_ref[0])
    g = _dotT(x, wg_ref[0])
    act = up * (g * jax.nn.sigmoid(g))
    part = jax.lax.dot_general(act, wd_ref[0], (((1,), (1,)), ((), ())),
                               preferred_element_type=jnp.float32)
    cols = jax.lax.broadcasted_iota(jnp.int32, (TOKENS, NUM_EXPERTS), 1)
    s = jnp.sum(jnp.where(cols == j, scale_ref[...], 0.0), axis=1,
                keepdims=True)
    out_ref[...] += part * s


def _shared_body(x_ref, wu_ref, wg_ref, wd_ref, routed_ref, out_ref):
    h = pl.program_id(0)

    @pl.when(h == 0)
    def _init():
        out_ref[...] = routed_ref[...]

    x = x_ref[...]
    up = _dotT(x, wu_ref[...])
    g = _dotT(x, wg_ref[...])
    act = up * (g * jax.nn.sigmoid(g))
    part = jax.lax.dot_general(act, wd_ref[...], (((1,), (1,)), ((), ())),
                               preferred_element_type=jnp.float32)
    out_ref[...] += part


@jax.jit
def kernel(x, Wr, Wu, Wg, Wd, Wu_s, Wg_s, Wd_s):
    routed = pl.pallas_call(
        _routed_body,
        grid=(N_ROUTED, NH),
        in_specs=[
            pl.BlockSpec((TOKENS, DIM), lambda j, h: (0, 0)),
            pl.BlockSpec((NUM_EXPERTS, DIM), lambda j, h: (0, 0)),
            pl.BlockSpec((1, HB, DIM), lambda j, h: (j, h, 0)),
            pl.BlockSpec((1, HB, DIM), lambda j, h: (j, h, 0)),
            pl.BlockSpec((1, DIM, HB), lambda j, h: (j, 0, h)),
        ],
        out_specs=pl.BlockSpec((TOKENS, DIM), lambda j, h: (0, 0)),
        out_shape=jax.ShapeDtypeStruct((TOKENS, DIM), jnp.float32),
        scratch_shapes=[pltpu.VMEM((TOKENS, NUM_EXPERTS), jnp.float32)],
        compiler_params=pltpu.CompilerParams(
            dimension_semantics=("arbitrary", "arbitrary"),
        ),
    )(x, Wr, Wu, Wg, Wd)

    out = pl.pallas_call(
        _shared_body,
        grid=(NH,),
        in_specs=[
            pl.BlockSpec((TOKENS, DIM), lambda h: (0, 0)),
            pl.BlockSpec((HB, DIM), lambda h: (h, 0)),
            pl.BlockSpec((HB, DIM), lambda h: (h, 0)),
            pl.BlockSpec((DIM, HB), lambda h: (0, h)),
            pl.BlockSpec((TOKENS, DIM), lambda h: (0, 0)),
        ],
        out_specs=pl.BlockSpec((TOKENS, DIM), lambda h: (0, 0)),
        out_shape=jax.ShapeDtypeStruct((TOKENS, DIM), jnp.float32),
        compiler_params=pltpu.CompilerParams(
            dimension_semantics=("arbitrary",),
        ),
    )(x, Wu_s, Wg_s, Wd_s, routed)
    return out
